# baseline (speedup 1.0000x reference)
import functools
import math

import jax
import jax.numpy as jnp
from jax import lax
from jax.experimental import pallas as pl
from jax.experimental.pallas import tpu as pltpu
from jax.experimental.pallas import tpu_sc as plsc

D_MODEL = 64
SCALE = math.sqrt(D_MODEL)
NUM_CORES = 2
NUM_SUBCORES = 16
NUM_WORKERS = NUM_CORES * NUM_SUBCORES
LANES = 16

VOCAB = 1000000
RBLK = 16000
SEQ = 200
BATCH = 4096
BSLICE = BATCH // NUM_WORKERS
NSTEP = SEQ // 2


def _repack_body(lutT_ref, lp_ref):
    lp_ref[:, 0:D_MODEL] = lutT_ref[...].T * SCALE


def _repack(lutT):
    return pl.pallas_call(
        _repack_body,
        grid=((VOCAB + RBLK - 1) // RBLK,),
        in_specs=[pl.BlockSpec((D_MODEL, RBLK), lambda i: (0, i))],
        out_specs=pl.BlockSpec((RBLK, 128), lambda i: (i, 0)),
        out_shape=jax.ShapeDtypeStruct((VOCAB, 128), jnp.float32),
    )(lutT)


NBUF = 4


def _gather_body(xT_hbm, lp_hbm, out_hbm, idx_v,
                 rows0, rows1, rows2, rows3, t0, t1,
                 gsem0, gsem1, gsem2, gsem3, wsem0, wsem1):
    wid = lax.axis_index("s") * NUM_CORES + lax.axis_index("c")
    b0 = wid * BSLICE

    rows = (rows0, rows1, rows2, rows3)
    tbuf = (t0, t1)
    gsem = (gsem0, gsem1, gsem2, gsem3)
    wsem = (wsem0, wsem1)

    pltpu.sync_copy(xT_hbm.at[:, pl.ds(b0, BSLICE)], idx_v)

    def gather(s, rb):
        pltpu.async_copy(lp_hbm.at[idx_v.at[s]], rows[rb], gsem[rb])

    def gather_wait(rb):
        pltpu.make_async_copy(lp_hbm.at[idx_v.at[0]], rows[rb], gsem[rb]).wait()

    lane_iota = lax.iota(jnp.int32, LANES)
    ridx_tab = [bg * LANES + lane_iota for bg in range(BSLICE // LANES)]

    def permute(rb, tb2):
        rv = rows[rb]
        tb = tbuf[tb2]

        @plsc.parallel_loop(0, D_MODEL, unroll=4)
        def _(d):
            dvec = jnp.zeros((LANES,), jnp.int32) + d
            for bg in range(BSLICE // LANES):
                vec = plsc.load_gather(rv, [ridx_tab[bg], dvec])
                tb[d, pl.ds(bg * LANES, LANES)] = vec

    def write(s, tb2):
        pltpu.async_copy(
            tbuf[tb2], out_hbm.at[s, :, pl.ds(b0, BSLICE)], wsem[tb2]
        )

    def write_wait(tb2):
        pltpu.make_async_copy(
            tbuf[tb2], out_hbm.at[0, :, pl.ds(b0, BSLICE)], wsem[tb2]
        ).wait()

    def step(s, rb, first, last):
        gather_wait(rb)
        if not last:
            gather(s + NBUF - 1, (rb + NBUF - 1) % NBUF)
        tb2 = rb % 2
        if not first:
            write_wait(tb2)
        permute(rb, tb2)
        write(s, tb2)

    for s in range(NBUF - 1):
        gather(s, s)

    step(0, 0, first=True, last=False)
    step(1, 1, first=True, last=False)
    step(2, 2, first=False, last=False)
    step(3, 3, first=False, last=False)

    def quad(q, c):
        s = 4 * q
        for rb in range(4):
            step(s + rb, rb, first=False, last=False)
        return c

    lax.fori_loop(1, SEQ // 4 - 1, quad, 0)

    step(SEQ - 4, 0, first=False, last=False)
    step(SEQ - 3, 1, first=False, last=True)
    step(SEQ - 2, 2, first=False, last=True)
    step(SEQ - 1, 3, first=False, last=True)

    write_wait(0)
    write_wait(1)


def _gather(xT, lp):
    mesh = plsc.VectorSubcoreMesh(
        core_axis_name="c",
        subcore_axis_name="s",
        num_cores=NUM_CORES,
        num_subcores=NUM_SUBCORES,
    )
    return pl.kernel(
        _gather_body,
        out_type=jax.ShapeDtypeStruct((SEQ, D_MODEL, BATCH), jnp.float32),
        mesh=mesh,
        scratch_types=[
            pltpu.VMEM((SEQ, BSLICE), jnp.int32),
            pltpu.VMEM((BSLICE, 128), jnp.float32),
            pltpu.VMEM((BSLICE, 128), jnp.float32),
            pltpu.VMEM((BSLICE, 128), jnp.float32),
            pltpu.VMEM((BSLICE, 128), jnp.float32),
            pltpu.VMEM((D_MODEL, BSLICE), jnp.float32),
            pltpu.VMEM((D_MODEL, BSLICE), jnp.float32),
            pltpu.SemaphoreType.DMA,
            pltpu.SemaphoreType.DMA,
            pltpu.SemaphoreType.DMA,
            pltpu.SemaphoreType.DMA,
            pltpu.SemaphoreType.DMA,
            pltpu.SemaphoreType.DMA,
        ],
        compiler_params=pltpu.CompilerParams(
            use_tc_tiling_on_sc=True, needs_layout_passes=False
        ),
    )(xT, lp)


def kernel(x, lut):
    lp = _repack(lut.T)
    outT = _gather(x.T, lp)
    return jnp.transpose(outT, (2, 0, 1))


# speedup vs baseline: 1.7864x; 1.7864x over previous
import functools
import math

import jax
import jax.numpy as jnp
from jax import lax
from jax.experimental import pallas as pl
from jax.experimental.pallas import tpu as pltpu
from jax.experimental.pallas import tpu_sc as plsc

D_MODEL = 64
SCALE = math.sqrt(D_MODEL)
NUM_CORES = 2
NUM_SUBCORES = 16
NUM_WORKERS = NUM_CORES * NUM_SUBCORES
LANES = 16

VOCAB = 1000000
RBLK = 16000
SEQ = 200
BATCH = 4096
BSLICE = BATCH // NUM_WORKERS
NSTEP = SEQ // 2


def _repack_body(lutT_ref, lp_ref):
    lp_ref[:, 0:D_MODEL] = lutT_ref[...].T * SCALE


def _repack(lutT):
    return pl.pallas_call(
        _repack_body,
        grid=((VOCAB + RBLK - 1) // RBLK,),
        in_specs=[pl.BlockSpec((D_MODEL, RBLK), lambda i: (0, i))],
        out_specs=pl.BlockSpec((RBLK, 128), lambda i: (i, 0)),
        out_shape=jax.ShapeDtypeStruct((VOCAB, 128), jnp.float32),
    )(lutT)


NBUF = 4


def _gather_body(xT_hbm, lp_hbm, out_hbm, idx_v,
                 rows0, rows1, rows2, rows3, t0, t1,
                 gsem0, gsem1, gsem2, gsem3, wsem0, wsem1):
    wid = lax.axis_index("s") * NUM_CORES + lax.axis_index("c")
    b0 = wid * BSLICE

    rows = (rows0, rows1, rows2, rows3)
    tbuf = (t0, t1)
    gsem = (gsem0, gsem1, gsem2, gsem3)
    wsem = (wsem0, wsem1)

    pltpu.sync_copy(xT_hbm.at[:, pl.ds(b0, BSLICE)], idx_v)

    def gather(s, rb):
        pltpu.async_copy(lp_hbm.at[idx_v.at[s]], rows[rb], gsem[rb])

    def gather_wait(rb):
        pltpu.make_async_copy(lp_hbm.at[idx_v.at[0]], rows[rb], gsem[rb]).wait()

    lane_iota = lax.iota(jnp.int32, LANES)
    row_tab = [bg * LANES + lane_iota for bg in range(BSLICE // LANES)]

    def permute(rb, tb2):
        rv = rows[rb]
        tb = tbuf[tb2]

        @plsc.parallel_loop(0, LANES, unroll=2)
        def _(d0):
            diag = lax.bitwise_and(d0 + lane_iota, LANES - 1)
            for dg in range(D_MODEL // LANES):
                col = dg * LANES + diag
                for bg in range(BSLICE // LANES):
                    vec = plsc.load_gather(rv, [row_tab[bg], col])
                    plsc.store_scatter(tb, [col, row_tab[bg]], vec)

    def write(s, tb2):
        pltpu.async_copy(
            tbuf[tb2], out_hbm.at[s, :, pl.ds(b0, BSLICE)], wsem[tb2]
        )

    def write_wait(tb2):
        pltpu.make_async_copy(
            tbuf[tb2], out_hbm.at[0, :, pl.ds(b0, BSLICE)], wsem[tb2]
        ).wait()

    def step(s, rb, first, last):
        gather_wait(rb)
        if not last:
            gather(s + NBUF - 1, (rb + NBUF - 1) % NBUF)
        tb2 = rb % 2
        if not first:
            write_wait(tb2)
        permute(rb, tb2)
        write(s, tb2)

    for s in range(NBUF - 1):
        gather(s, s)

    step(0, 0, first=True, last=False)
    step(1, 1, first=True, last=False)
    step(2, 2, first=False, last=False)
    step(3, 3, first=False, last=False)

    def quad(q, c):
        s = 4 * q
        for rb in range(4):
            step(s + rb, rb, first=False, last=False)
        return c

    lax.fori_loop(1, SEQ // 4 - 1, quad, 0)

    step(SEQ - 4, 0, first=False, last=False)
    step(SEQ - 3, 1, first=False, last=True)
    step(SEQ - 2, 2, first=False, last=True)
    step(SEQ - 1, 3, first=False, last=True)

    write_wait(0)
    write_wait(1)


def _gather(xT, lp):
    mesh = plsc.VectorSubcoreMesh(
        core_axis_name="c",
        subcore_axis_name="s",
        num_cores=NUM_CORES,
        num_subcores=NUM_SUBCORES,
    )
    return pl.kernel(
        _gather_body,
        out_type=jax.ShapeDtypeStruct((SEQ, D_MODEL, BATCH), jnp.float32),
        mesh=mesh,
        scratch_types=[
            pltpu.VMEM((SEQ, BSLICE), jnp.int32),
            pltpu.VMEM((BSLICE, 128), jnp.float32),
            pltpu.VMEM((BSLICE, 128), jnp.float32),
            pltpu.VMEM((BSLICE, 128), jnp.float32),
            pltpu.VMEM((BSLICE, 128), jnp.float32),
            pltpu.VMEM((D_MODEL, BSLICE), jnp.float32),
            pltpu.VMEM((D_MODEL, BSLICE), jnp.float32),
            pltpu.SemaphoreType.DMA,
            pltpu.SemaphoreType.DMA,
            pltpu.SemaphoreType.DMA,
            pltpu.SemaphoreType.DMA,
            pltpu.SemaphoreType.DMA,
            pltpu.SemaphoreType.DMA,
        ],
        compiler_params=pltpu.CompilerParams(
            use_tc_tiling_on_sc=True, needs_layout_passes=False
        ),
    )(xT, lp)


def kernel(x, lut):
    lp = _repack(lut.T)
    outT = _gather(x.T, lp)
    return jnp.transpose(outT, (2, 0, 1))


# speedup vs baseline: 1.8074x; 1.0117x over previous
import functools
import math

import jax
import jax.numpy as jnp
from jax import lax
from jax.experimental import pallas as pl
from jax.experimental.pallas import tpu as pltpu
from jax.experimental.pallas import tpu_sc as plsc

D_MODEL = 64
SCALE = math.sqrt(D_MODEL)
NUM_CORES = 2
NUM_SUBCORES = 16
NUM_WORKERS = NUM_CORES * NUM_SUBCORES
LANES = 16

VOCAB = 1000000
RBLK = 32000
SEQ = 200
BATCH = 4096
BSLICE = BATCH // NUM_WORKERS
NSTEP = SEQ // 2


def _repack_body(lutT_ref, lp_ref):
    lp_ref[:, 0:D_MODEL] = lutT_ref[...].T * SCALE


def _repack(lutT):
    return pl.pallas_call(
        _repack_body,
        grid=((VOCAB + RBLK - 1) // RBLK,),
        in_specs=[pl.BlockSpec((D_MODEL, RBLK), lambda i: (0, i))],
        out_specs=pl.BlockSpec((RBLK, 128), lambda i: (i, 0)),
        out_shape=jax.ShapeDtypeStruct((VOCAB, 128), jnp.float32),
    )(lutT)


NBUF = 4


def _gather_body(xT_hbm, lp_hbm, out_hbm, idx_v,
                 rows0, rows1, rows2, rows3, t0, t1,
                 gsem0, gsem1, gsem2, gsem3, wsem0, wsem1):
    wid = lax.axis_index("s") * NUM_CORES + lax.axis_index("c")
    b0 = wid * BSLICE

    rows = (rows0, rows1, rows2, rows3)
    tbuf = (t0, t1)
    gsem = (gsem0, gsem1, gsem2, gsem3)
    wsem = (wsem0, wsem1)

    pltpu.sync_copy(xT_hbm.at[:, pl.ds(b0, BSLICE)], idx_v)

    def gather(s, rb):
        pltpu.async_copy(lp_hbm.at[idx_v.at[s]], rows[rb], gsem[rb])

    def gather_wait(rb):
        pltpu.make_async_copy(lp_hbm.at[idx_v.at[0]], rows[rb], gsem[rb]).wait()

    lane_iota = lax.iota(jnp.int32, LANES)
    row_tab = [bg * LANES + lane_iota for bg in range(BSLICE // LANES)]

    def permute(rb, tb2):
        rv = rows[rb]
        tb = tbuf[tb2]

        @plsc.parallel_loop(0, LANES, unroll=2)
        def _(d0):
            diag = lax.bitwise_and(d0 + lane_iota, LANES - 1)
            for dg in range(D_MODEL // LANES):
                col = dg * LANES + diag
                for bg in range(BSLICE // LANES):
                    vec = plsc.load_gather(rv, [row_tab[bg], col])
                    plsc.store_scatter(tb, [col, row_tab[bg]], vec)

    def write(s, tb2):
        pltpu.async_copy(
            tbuf[tb2], out_hbm.at[s, :, pl.ds(b0, BSLICE)], wsem[tb2]
        )

    def write_wait(tb2):
        pltpu.make_async_copy(
            tbuf[tb2], out_hbm.at[0, :, pl.ds(b0, BSLICE)], wsem[tb2]
        ).wait()

    def step(s, rb, first, last):
        gather_wait(rb)
        if not last:
            gather(s + NBUF - 1, (rb + NBUF - 1) % NBUF)
        tb2 = rb % 2
        if not first:
            write_wait(tb2)
        permute(rb, tb2)
        write(s, tb2)

    for s in range(NBUF - 1):
        gather(s, s)

    step(0, 0, first=True, last=False)
    step(1, 1, first=True, last=False)
    step(2, 2, first=False, last=False)
    step(3, 3, first=False, last=False)

    def quad(q, c):
        s = 4 * q
        for rb in range(4):
            step(s + rb, rb, first=False, last=False)
        return c

    lax.fori_loop(1, SEQ // 4 - 1, quad, 0)

    step(SEQ - 4, 0, first=False, last=False)
    step(SEQ - 3, 1, first=False, last=True)
    step(SEQ - 2, 2, first=False, last=True)
    step(SEQ - 1, 3, first=False, last=True)

    write_wait(0)
    write_wait(1)


def _gather(xT, lp):
    mesh = plsc.VectorSubcoreMesh(
        core_axis_name="c",
        subcore_axis_name="s",
        num_cores=NUM_CORES,
        num_subcores=NUM_SUBCORES,
    )
    return pl.kernel(
        _gather_body,
        out_type=jax.ShapeDtypeStruct((SEQ, D_MODEL, BATCH), jnp.float32),
        mesh=mesh,
        scratch_types=[
            pltpu.VMEM((SEQ, BSLICE), jnp.int32),
            pltpu.VMEM((BSLICE, 128), jnp.float32),
            pltpu.VMEM((BSLICE, 128), jnp.float32),
            pltpu.VMEM((BSLICE, 128), jnp.float32),
            pltpu.VMEM((BSLICE, 128), jnp.float32),
            pltpu.VMEM((D_MODEL, BSLICE), jnp.float32),
            pltpu.VMEM((D_MODEL, BSLICE), jnp.float32),
            pltpu.SemaphoreType.DMA,
            pltpu.SemaphoreType.DMA,
            pltpu.SemaphoreType.DMA,
            pltpu.SemaphoreType.DMA,
            pltpu.SemaphoreType.DMA,
            pltpu.SemaphoreType.DMA,
        ],
        compiler_params=pltpu.CompilerParams(
            use_tc_tiling_on_sc=True, needs_layout_passes=False
        ),
    )(xT, lp)


def kernel(x, lut):
    lp = _repack(lut.T)
    outT = _gather(x.T, lp)
    return jnp.transpose(outT, (2, 0, 1))
